# baseline (speedup 1.0000x reference)
import functools

import numpy as np

import jax
import jax.numpy as jnp
from jax import lax
from jax.experimental import pallas as pl
from jax.experimental.pallas import tpu as pltpu
from jax.experimental.pallas import tpu_sc as plsc

_EPS = 1e-15
_MAXNORM = 1.0 - 4e-3

_NUM_CORES = 2
_NUM_SUBCORES = 16
_NW = _NUM_CORES * _NUM_SUBCORES
_CH = 128
_SUP = 16


def _artanh(z):
    z = jnp.clip(z, -1.0 + 1e-7, 1.0 - 1e-7)
    return 0.5 * jnp.log((1.0 + z) / (1.0 - z))


def _row_norm(v):
    return jnp.maximum(jnp.sqrt(jnp.sum(v * v, axis=-1, keepdims=True)), _EPS)


def _proj(v):
    n = _row_norm(v)
    return jnp.where(n > _MAXNORM, v / n * _MAXNORM, v)


def _phase_a_body(x_ref, wt_ref, b_ref, o_ref):
    x = x_ref[...]
    mx = jnp.dot(x, wt_ref[...], preferred_element_type=jnp.float32)
    x_norm = _row_norm(x)
    mx_norm = _row_norm(mx)
    res = jnp.tanh(mx_norm / x_norm * _artanh(x_norm)) * mx / mx_norm
    res = jnp.where(jnp.all(mx == 0.0, axis=-1, keepdims=True), 0.0, res)
    h = _proj(res)
    b = b_ref[...]
    bn = _row_norm(b)
    y = _proj(jnp.tanh(bn) * b / bn)
    x2 = jnp.sum(h * h, axis=-1, keepdims=True)
    y2 = jnp.sum(y * y, axis=-1, keepdims=True)
    xy = jnp.sum(h * y, axis=-1, keepdims=True)
    num = (1.0 + 2.0 * xy + y2) * h + (1.0 - x2) * y
    den = 1.0 + 2.0 * xy + x2 * y2
    h = _proj(num / jnp.maximum(den, _EPS))
    pn = _row_norm(h)
    o_ref[...] = _artanh(pn) * h / pn


def _phase_c_body(a_ref, b_ref, o_ref):
    v = a_ref[...] + b_ref[...]
    sn = _row_norm(v)
    o1 = _proj(jnp.tanh(sn) * v / sn)
    pn = _row_norm(o1)
    xt = jnp.maximum(_artanh(pn) * o1 / pn, 0.0)
    un = _row_norm(xt)
    o_ref[...] = _proj(jnp.tanh(un) * xt / un)


def _sc_segment_sum(xt, col_p, row_p, n_nodes, n_pad, nch0, nch1):
    d = xt.shape[1]
    nch = max(nch0, nch1)
    mesh = plsc.VectorSubcoreMesh(core_axis_name="c", subcore_axis_name="s")

    @functools.partial(
        pl.kernel,
        mesh=mesh,
        out_type=jax.ShapeDtypeStruct((_NUM_CORES, n_pad, d), jnp.float32),
        scratch_types=[
            pltpu.VMEM((nch, _CH), jnp.int32),
            pltpu.VMEM((2, 1, _CH), jnp.int32),
            pltpu.VMEM((2, _CH, d), jnp.float32),
            pltpu.VMEM_SHARED((n_pad, d), jnp.float32),
            pltpu.SemaphoreType.DMA((2,)),
            pltpu.SemaphoreType.DMA((2,)),
        ],
    )
    def seg(xt_hbm, col_hbm, row_hbm, out_hbm, col_v, ridx, gbuf, acc,
            gsem, rsem):
        cid = lax.axis_index("c")
        sid = lax.axis_index("s")
        wid = sid * _NUM_CORES + cid
        nch_c = jnp.where(cid == 0, nch0, nch1)

        buf0 = gbuf.at[0]

        def _zero_row(i, carry):
            for k in range(d // 16):
                gbuf[0, i, pl.ds(k * 16, 16)] = jnp.zeros((16,), jnp.float32)
            return carry

        lax.fori_loop(0, _CH, _zero_row, 0)
        rows_per_tile = n_pad // _NUM_SUBCORES
        base = sid * rows_per_tile
        off = 0
        while off < rows_per_tile:
            nr = min(_CH, rows_per_tile - off)
            pltpu.sync_copy(buf0.at[pl.ds(0, nr)],
                            acc.at[pl.ds(base + off, nr)])
            off += nr

        pltpu.sync_copy(col_hbm.at[wid], col_v)
        pltpu.async_copy(row_hbm.at[wid].at[0], ridx.at[0], rsem.at[0])
        pltpu.async_copy(row_hbm.at[wid].at[1], ridx.at[1], rsem.at[1])
        plsc.subcore_barrier()

        pltpu.async_copy(xt_hbm.at[col_v.at[0]], gbuf.at[0], gsem.at[0])

        def _step(j, carry):
            b = lax.rem(j, 2)
            nb = 1 - b
            pltpu.make_async_copy(
                xt_hbm.at[col_v.at[j]], gbuf.at[b], gsem.at[b]).wait()

            @pl.when(j + 1 < nch_c)
            def _():
                pltpu.async_copy(
                    xt_hbm.at[col_v.at[j + 1]], gbuf.at[nb], gsem.at[nb])

            pltpu.make_async_copy(
                row_hbm.at[wid].at[j], ridx.at[b], rsem.at[b]).wait()
            pltpu.sync_copy(gbuf.at[b], acc.at[ridx.at[b, 0]], add=True)

            @pl.when(j + 2 < nch_c)
            def _():
                pltpu.async_copy(
                    row_hbm.at[wid].at[j + 2], ridx.at[b], rsem.at[b])

            return carry

        lax.fori_loop(0, nch_c, _step, 0)
        plsc.subcore_barrier()

        pltpu.sync_copy(
            acc.at[pl.ds(base, rows_per_tile)],
            out_hbm.at[cid].at[pl.ds(base, rows_per_tile)],
        )

    return seg(xt, col_p, row_p)


def kernel(x, edge_index, W, bias):
    n, d = x.shape
    e = edge_index.shape[1]

    blk = 1000
    grid = (n // blk,)
    xt = pl.pallas_call(
        _phase_a_body,
        grid=grid,
        in_specs=[
            pl.BlockSpec((blk, d), lambda i: (i, 0)),
            pl.BlockSpec((d, d), lambda i: (0, 0)),
            pl.BlockSpec((1, d), lambda i: (0, 0)),
        ],
        out_specs=pl.BlockSpec((blk, d), lambda i: (i, 0)),
        out_shape=jax.ShapeDtypeStruct((n, d), jnp.float32),
    )(x, W.T, bias.reshape(1, d))

    total_chunks = -(-e // _CH)
    pair_total = -(-total_chunks // _NUM_SUBCORES)
    nch1 = max(2, min(pair_total - 2, int(round(pair_total * 0.25))))
    nch0 = pair_total - nch1
    e_pad = _NUM_SUBCORES * pair_total * _CH
    n_pad = -(-(n + 1) // (8 * _NUM_SUBCORES)) * (8 * _NUM_SUBCORES)
    row = edge_index[0]
    col = edge_index[1]
    pad_dst = n + (jnp.arange(e_pad - e, dtype=jnp.int32) % (n_pad - n))
    col_flat = jnp.concatenate(
        [col, jnp.zeros((e_pad - e,), jnp.int32)]).reshape(-1, _CH)
    row_flat = jnp.concatenate([row, pad_dst]).reshape(-1, _CH)
    blocks = [nch0 if (w % _NUM_CORES) == 0 else nch1 for w in range(_NW)]
    starts = np.concatenate([[0], np.cumsum(blocks)[:-1]])
    nch_max = max(nch0, nch1)
    cmap = np.zeros((_NW, nch_max), np.int32)
    for w in range(_NW):
        cmap[w, :blocks[w]] = starts[w] + np.arange(blocks[w])
    cmap_j = jnp.asarray(cmap)
    col_p = col_flat[cmap_j]
    row_p = row_flat[cmap_j][:, :, None, :]

    parts = _sc_segment_sum(xt, col_p, row_p, n, n_pad, nch0, nch1)

    parts = parts[:, :n, :]
    out = pl.pallas_call(
        _phase_c_body,
        grid=grid,
        in_specs=[
            pl.BlockSpec((blk, d), lambda i: (i, 0)),
            pl.BlockSpec((blk, d), lambda i: (i, 0)),
        ],
        out_specs=pl.BlockSpec((blk, d), lambda i: (i, 0)),
        out_shape=jax.ShapeDtypeStruct((n, d), jnp.float32),
    )(parts[0], parts[1])
    return out


# speedup vs baseline: 1.2232x; 1.2232x over previous
import functools

import numpy as np

import jax
import jax.numpy as jnp
from jax import lax
from jax.experimental import pallas as pl
from jax.experimental.pallas import tpu as pltpu
from jax.experimental.pallas import tpu_sc as plsc

_EPS = 1e-15
_MAXNORM = 1.0 - 4e-3

_NUM_CORES = 2
_NUM_SUBCORES = 16
_NW = _NUM_CORES * _NUM_SUBCORES
_CH = 128
_SUP = 16


def _artanh(z):
    z = jnp.clip(z, -1.0 + 1e-7, 1.0 - 1e-7)
    return 0.5 * jnp.log((1.0 + z) / (1.0 - z))


def _row_norm(v):
    return jnp.maximum(jnp.sqrt(jnp.sum(v * v, axis=-1, keepdims=True)), _EPS)


def _proj(v):
    n = _row_norm(v)
    return jnp.where(n > _MAXNORM, v / n * _MAXNORM, v)


def _phase_a_body(x_ref, wt_ref, b_ref, o_ref):
    x = x_ref[...]
    mx = jnp.dot(x, wt_ref[...], preferred_element_type=jnp.float32)
    x_norm = _row_norm(x)
    mx_norm = _row_norm(mx)
    res = jnp.tanh(mx_norm / x_norm * _artanh(x_norm)) * mx / mx_norm
    res = jnp.where(jnp.all(mx == 0.0, axis=-1, keepdims=True), 0.0, res)
    h = _proj(res)
    b = b_ref[...]
    bn = _row_norm(b)
    y = _proj(jnp.tanh(bn) * b / bn)
    x2 = jnp.sum(h * h, axis=-1, keepdims=True)
    y2 = jnp.sum(y * y, axis=-1, keepdims=True)
    xy = jnp.sum(h * y, axis=-1, keepdims=True)
    num = (1.0 + 2.0 * xy + y2) * h + (1.0 - x2) * y
    den = 1.0 + 2.0 * xy + x2 * y2
    h = _proj(num / jnp.maximum(den, _EPS))
    pn = _row_norm(h)
    o_ref[...] = _artanh(pn) * h / pn


def _phase_c_body(a_ref, b_ref, o_ref):
    v = a_ref[...] + b_ref[...]
    sn = _row_norm(v)
    o1 = _proj(jnp.tanh(sn) * v / sn)
    pn = _row_norm(o1)
    xt = jnp.maximum(_artanh(pn) * o1 / pn, 0.0)
    un = _row_norm(xt)
    o_ref[...] = _proj(jnp.tanh(un) * xt / un)


def _sc_segment_sum(xt, col_p, row_p, n_nodes, n_pad, nch0, nch1):
    d = xt.shape[1]
    nch = max(nch0, nch1)
    mesh = plsc.VectorSubcoreMesh(core_axis_name="c", subcore_axis_name="s")

    @functools.partial(
        pl.kernel,
        mesh=mesh,
        out_type=jax.ShapeDtypeStruct((_NUM_CORES, n_pad, d), jnp.float32),
        scratch_types=[
            pltpu.VMEM((nch, _CH), jnp.int32),
            pltpu.VMEM((2, 1, _CH), jnp.int32),
            pltpu.VMEM((2, _CH, d), jnp.float32),
            pltpu.VMEM_SHARED((n_pad, d), jnp.float32),
            pltpu.SemaphoreType.DMA((2,)),
            pltpu.SemaphoreType.DMA((2,)),
        ],
    )
    def seg(xt_hbm, col_hbm, row_hbm, out_hbm, col_v, ridx, gbuf, acc,
            gsem, rsem):
        cid = lax.axis_index("c")
        sid = lax.axis_index("s")
        wid = sid * _NUM_CORES + cid
        nch_c = jnp.where(cid == 0, nch0, nch1)

        buf0 = gbuf.at[0]

        def _zero_row(i, carry):
            for k in range(d // 16):
                gbuf[0, i, pl.ds(k * 16, 16)] = jnp.zeros((16,), jnp.float32)
            return carry

        lax.fori_loop(0, _CH, _zero_row, 0)
        rows_per_tile = n_pad // _NUM_SUBCORES
        base = sid * rows_per_tile
        off = 0
        while off < rows_per_tile:
            nr = min(_CH, rows_per_tile - off)
            pltpu.sync_copy(buf0.at[pl.ds(0, nr)],
                            acc.at[pl.ds(base + off, nr)])
            off += nr

        pltpu.sync_copy(col_hbm.at[wid], col_v)
        pltpu.async_copy(row_hbm.at[wid].at[0], ridx.at[0], rsem.at[0])
        pltpu.async_copy(row_hbm.at[wid].at[1], ridx.at[1], rsem.at[1])
        plsc.subcore_barrier()

        pltpu.async_copy(xt_hbm.at[col_v.at[0]], gbuf.at[0], gsem.at[0])

        def _step(j, carry):
            b = lax.rem(j, 2)
            nb = 1 - b
            pltpu.make_async_copy(
                xt_hbm.at[col_v.at[j]], gbuf.at[b], gsem.at[b]).wait()

            @pl.when(j + 1 < nch_c)
            def _():
                pltpu.async_copy(
                    xt_hbm.at[col_v.at[j + 1]], gbuf.at[nb], gsem.at[nb])

            pltpu.make_async_copy(
                row_hbm.at[wid].at[j], ridx.at[b], rsem.at[b]).wait()
            pltpu.sync_copy(gbuf.at[b], acc.at[ridx.at[b, 0]], add=True)

            @pl.when(j + 2 < nch_c)
            def _():
                pltpu.async_copy(
                    row_hbm.at[wid].at[j + 2], ridx.at[b], rsem.at[b])

            return carry

        lax.fori_loop(0, nch_c, _step, 0)
        plsc.subcore_barrier()

        pltpu.sync_copy(
            acc.at[pl.ds(base, rows_per_tile)],
            out_hbm.at[cid].at[pl.ds(base, rows_per_tile)],
        )

    return seg(xt, col_p, row_p)


def kernel(x, edge_index, W, bias):
    n, d = x.shape
    e = edge_index.shape[1]

    blk = 1000
    grid = (n // blk,)
    xt = pl.pallas_call(
        _phase_a_body,
        grid=grid,
        in_specs=[
            pl.BlockSpec((blk, d), lambda i: (i, 0)),
            pl.BlockSpec((d, d), lambda i: (0, 0)),
            pl.BlockSpec((1, d), lambda i: (0, 0)),
        ],
        out_specs=pl.BlockSpec((blk, d), lambda i: (i, 0)),
        out_shape=jax.ShapeDtypeStruct((n, d), jnp.float32),
    )(x, W.T, bias.reshape(1, d))

    total_chunks = -(-e // _CH)
    pair_total = -(-total_chunks // _NUM_SUBCORES)
    nch1 = max(2, min(pair_total - 2, int(round(pair_total * 0.37))))
    nch0 = pair_total - nch1
    e_pad = _NUM_SUBCORES * pair_total * _CH
    n_pad = -(-(n + 1) // (8 * _NUM_SUBCORES)) * (8 * _NUM_SUBCORES)
    row = edge_index[0]
    col = edge_index[1]
    pad_dst = n + (jnp.arange(e_pad - e, dtype=jnp.int32) % (n_pad - n))
    col_flat = jnp.concatenate(
        [col, jnp.zeros((e_pad - e,), jnp.int32)]).reshape(-1, _CH)
    row_flat = jnp.concatenate([row, pad_dst]).reshape(-1, _CH)
    blocks = [nch0 if (w % _NUM_CORES) == 0 else nch1 for w in range(_NW)]
    starts = np.concatenate([[0], np.cumsum(blocks)[:-1]])
    nch_max = max(nch0, nch1)
    cmap = np.zeros((_NW, nch_max), np.int32)
    for w in range(_NW):
        cmap[w, :blocks[w]] = starts[w] + np.arange(blocks[w])
    cmap_j = jnp.asarray(cmap)
    col_p = col_flat[cmap_j]
    row_p = row_flat[cmap_j][:, :, None, :]

    parts = _sc_segment_sum(xt, col_p, row_p, n, n_pad, nch0, nch1)

    parts = parts[:, :n, :]
    out = pl.pallas_call(
        _phase_c_body,
        grid=grid,
        in_specs=[
            pl.BlockSpec((blk, d), lambda i: (i, 0)),
            pl.BlockSpec((blk, d), lambda i: (i, 0)),
        ],
        out_specs=pl.BlockSpec((blk, d), lambda i: (i, 0)),
        out_shape=jax.ShapeDtypeStruct((n, d), jnp.float32),
    )(parts[0], parts[1])
    return out


# speedup vs baseline: 1.2928x; 1.0568x over previous
import functools

import numpy as np

import jax
import jax.numpy as jnp
from jax import lax
from jax.experimental import pallas as pl
from jax.experimental.pallas import tpu as pltpu
from jax.experimental.pallas import tpu_sc as plsc

_EPS = 1e-15
_MAXNORM = 1.0 - 4e-3

_NUM_CORES = 2
_NUM_SUBCORES = 16
_NW = _NUM_CORES * _NUM_SUBCORES
_CH = 128
_SUP = 16


def _artanh(z):
    z = jnp.clip(z, -1.0 + 1e-7, 1.0 - 1e-7)
    return 0.5 * jnp.log((1.0 + z) / (1.0 - z))


def _row_norm(v):
    return jnp.maximum(jnp.sqrt(jnp.sum(v * v, axis=-1, keepdims=True)), _EPS)


def _proj(v):
    n = _row_norm(v)
    return jnp.where(n > _MAXNORM, v / n * _MAXNORM, v)


def _phase_a_body(x_ref, wt_ref, b_ref, o_ref):
    x = x_ref[...]
    mx = jnp.dot(x, wt_ref[...], preferred_element_type=jnp.float32)
    x_norm = _row_norm(x)
    mx_norm = _row_norm(mx)
    res = jnp.tanh(mx_norm / x_norm * _artanh(x_norm)) * mx / mx_norm
    res = jnp.where(jnp.all(mx == 0.0, axis=-1, keepdims=True), 0.0, res)
    h = _proj(res)
    b = b_ref[...]
    bn = _row_norm(b)
    y = _proj(jnp.tanh(bn) * b / bn)
    x2 = jnp.sum(h * h, axis=-1, keepdims=True)
    y2 = jnp.sum(y * y, axis=-1, keepdims=True)
    xy = jnp.sum(h * y, axis=-1, keepdims=True)
    num = (1.0 + 2.0 * xy + y2) * h + (1.0 - x2) * y
    den = 1.0 + 2.0 * xy + x2 * y2
    h = _proj(num / jnp.maximum(den, _EPS))
    pn = _row_norm(h)
    o_ref[...] = _artanh(pn) * h / pn


def _phase_c_body(a_ref, b_ref, o_ref):
    v = a_ref[...] + b_ref[...]
    sn = _row_norm(v)
    o1 = _proj(jnp.tanh(sn) * v / sn)
    pn = _row_norm(o1)
    xt = jnp.maximum(_artanh(pn) * o1 / pn, 0.0)
    un = _row_norm(xt)
    o_ref[...] = _proj(jnp.tanh(un) * xt / un)


def _sc_segment_sum(xt, col_p, row_p, n_nodes, n_pad, nch0, nch1):
    d = xt.shape[1]
    nch = max(nch0, nch1)
    mesh = plsc.VectorSubcoreMesh(core_axis_name="c", subcore_axis_name="s")

    @functools.partial(
        pl.kernel,
        mesh=mesh,
        out_type=jax.ShapeDtypeStruct((_NUM_CORES, n_pad, d), jnp.float32),
        scratch_types=[
            pltpu.VMEM((nch, _CH), jnp.int32),
            pltpu.VMEM((2, 1, _CH), jnp.int32),
            pltpu.VMEM((2, _CH, d), jnp.float32),
            pltpu.VMEM_SHARED((n_pad, d), jnp.float32),
            pltpu.SemaphoreType.DMA((2,)),
            pltpu.SemaphoreType.DMA((2,)),
        ],
    )
    def seg(xt_hbm, col_hbm, row_hbm, out_hbm, col_v, ridx, gbuf, acc,
            gsem, rsem):
        cid = lax.axis_index("c")
        sid = lax.axis_index("s")
        wid = sid * _NUM_CORES + cid
        nch_c = jnp.where(cid == 0, nch0, nch1)

        buf0 = gbuf.at[0]

        def _zero_row(i, carry):
            for k in range(d // 16):
                gbuf[0, i, pl.ds(k * 16, 16)] = jnp.zeros((16,), jnp.float32)
            return carry

        lax.fori_loop(0, _CH, _zero_row, 0)
        rows_per_tile = n_pad // _NUM_SUBCORES
        base = sid * rows_per_tile
        off = 0
        while off < rows_per_tile:
            nr = min(_CH, rows_per_tile - off)
            pltpu.sync_copy(buf0.at[pl.ds(0, nr)],
                            acc.at[pl.ds(base + off, nr)])
            off += nr

        pltpu.sync_copy(col_hbm.at[wid], col_v)
        pltpu.async_copy(row_hbm.at[wid].at[0], ridx.at[0], rsem.at[0])
        pltpu.async_copy(row_hbm.at[wid].at[1], ridx.at[1], rsem.at[1])
        plsc.subcore_barrier()

        pltpu.async_copy(xt_hbm.at[col_v.at[0]], gbuf.at[0], gsem.at[0])

        def _step(j, carry):
            b = lax.rem(j, 2)
            nb = 1 - b
            pltpu.make_async_copy(
                xt_hbm.at[col_v.at[j]], gbuf.at[b], gsem.at[b]).wait()

            @pl.when(j + 1 < nch_c)
            def _():
                pltpu.async_copy(
                    xt_hbm.at[col_v.at[j + 1]], gbuf.at[nb], gsem.at[nb])

            pltpu.make_async_copy(
                row_hbm.at[wid].at[j], ridx.at[b], rsem.at[b]).wait()
            pltpu.sync_copy(gbuf.at[b], acc.at[ridx.at[b, 0]], add=True)

            @pl.when(j + 2 < nch_c)
            def _():
                pltpu.async_copy(
                    row_hbm.at[wid].at[j + 2], ridx.at[b], rsem.at[b])

            return carry

        lax.fori_loop(0, nch_c, _step, 0)
        plsc.subcore_barrier()

        pltpu.sync_copy(
            acc.at[pl.ds(base, rows_per_tile)],
            out_hbm.at[cid].at[pl.ds(base, rows_per_tile)],
        )

    return seg(xt, col_p, row_p)


def kernel(x, edge_index, W, bias):
    n, d = x.shape
    e = edge_index.shape[1]

    blk = 1000
    grid = (n // blk,)
    xt = pl.pallas_call(
        _phase_a_body,
        grid=grid,
        in_specs=[
            pl.BlockSpec((blk, d), lambda i: (i, 0)),
            pl.BlockSpec((d, d), lambda i: (0, 0)),
            pl.BlockSpec((1, d), lambda i: (0, 0)),
        ],
        out_specs=pl.BlockSpec((blk, d), lambda i: (i, 0)),
        out_shape=jax.ShapeDtypeStruct((n, d), jnp.float32),
    )(x, W.T, bias.reshape(1, d))

    total_chunks = -(-e // _CH)
    pair_total = -(-total_chunks // _NUM_SUBCORES)
    nch1 = max(2, min(pair_total - 2, int(round(pair_total * 0.42))))
    nch0 = pair_total - nch1
    e_pad = _NUM_SUBCORES * pair_total * _CH
    n_pad = -(-(n + 1) // (8 * _NUM_SUBCORES)) * (8 * _NUM_SUBCORES)
    row = edge_index[0]
    col = edge_index[1]
    pad_dst = n + (jnp.arange(e_pad - e, dtype=jnp.int32) % (n_pad - n))
    col_flat = jnp.concatenate(
        [col, jnp.zeros((e_pad - e,), jnp.int32)]).reshape(-1, _CH)
    row_flat = jnp.concatenate([row, pad_dst]).reshape(-1, _CH)
    blocks = [nch0 if (w % _NUM_CORES) == 0 else nch1 for w in range(_NW)]
    starts = np.concatenate([[0], np.cumsum(blocks)[:-1]])
    nch_max = max(nch0, nch1)
    cmap = np.zeros((_NW, nch_max), np.int32)
    for w in range(_NW):
        cmap[w, :blocks[w]] = starts[w] + np.arange(blocks[w])
    cmap_j = jnp.asarray(cmap)
    col_p = col_flat[cmap_j]
    row_p = row_flat[cmap_j][:, :, None, :]

    parts = _sc_segment_sum(xt, col_p, row_p, n, n_pad, nch0, nch1)

    parts = parts[:, :n, :]
    out = pl.pallas_call(
        _phase_c_body,
        grid=grid,
        in_specs=[
            pl.BlockSpec((blk, d), lambda i: (i, 0)),
            pl.BlockSpec((blk, d), lambda i: (i, 0)),
        ],
        out_specs=pl.BlockSpec((blk, d), lambda i: (i, 0)),
        out_shape=jax.ShapeDtypeStruct((n, d), jnp.float32),
    )(parts[0], parts[1])
    return out


# speedup vs baseline: 1.3079x; 1.0117x over previous
import functools

import numpy as np

import jax
import jax.numpy as jnp
from jax import lax
from jax.experimental import pallas as pl
from jax.experimental.pallas import tpu as pltpu
from jax.experimental.pallas import tpu_sc as plsc

_EPS = 1e-15
_MAXNORM = 1.0 - 4e-3

_NUM_CORES = 2
_NUM_SUBCORES = 16
_NW = _NUM_CORES * _NUM_SUBCORES
_CH = 128
_SUP = 16


def _artanh(z):
    z = jnp.clip(z, -1.0 + 1e-7, 1.0 - 1e-7)
    return 0.5 * jnp.log((1.0 + z) / (1.0 - z))


def _row_norm(v):
    return jnp.maximum(jnp.sqrt(jnp.sum(v * v, axis=-1, keepdims=True)), _EPS)


def _proj(v):
    n = _row_norm(v)
    return jnp.where(n > _MAXNORM, v / n * _MAXNORM, v)


def _phase_a_body(x_ref, wt_ref, b_ref, o_ref):
    x = x_ref[...]
    mx = jnp.dot(x, wt_ref[...], preferred_element_type=jnp.float32)
    x_norm = _row_norm(x)
    mx_norm = _row_norm(mx)
    res = jnp.tanh(mx_norm / x_norm * _artanh(x_norm)) * mx / mx_norm
    res = jnp.where(jnp.all(mx == 0.0, axis=-1, keepdims=True), 0.0, res)
    h = _proj(res)
    b = b_ref[...]
    bn = _row_norm(b)
    y = _proj(jnp.tanh(bn) * b / bn)
    x2 = jnp.sum(h * h, axis=-1, keepdims=True)
    y2 = jnp.sum(y * y, axis=-1, keepdims=True)
    xy = jnp.sum(h * y, axis=-1, keepdims=True)
    num = (1.0 + 2.0 * xy + y2) * h + (1.0 - x2) * y
    den = 1.0 + 2.0 * xy + x2 * y2
    h = _proj(num / jnp.maximum(den, _EPS))
    pn = _row_norm(h)
    o_ref[...] = _artanh(pn) * h / pn


def _phase_c_body(a_ref, b_ref, o_ref):
    v = a_ref[...] + b_ref[...]
    sn = _row_norm(v)
    o1 = _proj(jnp.tanh(sn) * v / sn)
    pn = _row_norm(o1)
    xt = jnp.maximum(_artanh(pn) * o1 / pn, 0.0)
    un = _row_norm(xt)
    o_ref[...] = _proj(jnp.tanh(un) * xt / un)


def _sc_segment_sum(xt, col_p, row_p, n_nodes, n_pad, nch0, nch1):
    d = xt.shape[1]
    nch = max(nch0, nch1)
    mesh = plsc.VectorSubcoreMesh(core_axis_name="c", subcore_axis_name="s")

    @functools.partial(
        pl.kernel,
        mesh=mesh,
        out_type=jax.ShapeDtypeStruct((_NUM_CORES, n_pad, d), jnp.float32),
        scratch_types=[
            pltpu.VMEM((nch, _CH), jnp.int32),
            pltpu.VMEM((2, 1, _CH), jnp.int32),
            pltpu.VMEM((2, _CH, d), jnp.float32),
            pltpu.VMEM_SHARED((n_pad, d), jnp.float32),
            pltpu.SemaphoreType.DMA((2,)),
            pltpu.SemaphoreType.DMA((2,)),
        ],
    )
    def seg(xt_hbm, col_hbm, row_hbm, out_hbm, col_v, ridx, gbuf, acc,
            gsem, rsem):
        cid = lax.axis_index("c")
        sid = lax.axis_index("s")
        wid = sid * _NUM_CORES + cid
        nch_c = jnp.where(cid == 0, nch0, nch1)

        buf0 = gbuf.at[0]

        def _zero_row(i, carry):
            for k in range(d // 16):
                gbuf[0, i, pl.ds(k * 16, 16)] = jnp.zeros((16,), jnp.float32)
            return carry

        lax.fori_loop(0, _CH, _zero_row, 0)
        rows_per_tile = n_pad // _NUM_SUBCORES
        base = sid * rows_per_tile
        off = 0
        while off < rows_per_tile:
            nr = min(_CH, rows_per_tile - off)
            pltpu.sync_copy(buf0.at[pl.ds(0, nr)],
                            acc.at[pl.ds(base + off, nr)])
            off += nr

        pltpu.sync_copy(col_hbm.at[wid], col_v)
        pltpu.async_copy(row_hbm.at[wid].at[0], ridx.at[0], rsem.at[0])
        pltpu.async_copy(row_hbm.at[wid].at[1], ridx.at[1], rsem.at[1])
        plsc.subcore_barrier()

        pltpu.async_copy(xt_hbm.at[col_v.at[0]], gbuf.at[0], gsem.at[0])

        def _step(j, carry):
            b = lax.rem(j, 2)
            nb = 1 - b
            pltpu.make_async_copy(
                xt_hbm.at[col_v.at[j]], gbuf.at[b], gsem.at[b]).wait()

            @pl.when(j + 1 < nch_c)
            def _():
                pltpu.async_copy(
                    xt_hbm.at[col_v.at[j + 1]], gbuf.at[nb], gsem.at[nb])

            pltpu.make_async_copy(
                row_hbm.at[wid].at[j], ridx.at[b], rsem.at[b]).wait()
            pltpu.sync_copy(gbuf.at[b], acc.at[ridx.at[b, 0]], add=True)

            @pl.when(j + 2 < nch_c)
            def _():
                pltpu.async_copy(
                    row_hbm.at[wid].at[j + 2], ridx.at[b], rsem.at[b])

            return carry

        lax.fori_loop(0, nch_c, _step, 0)
        plsc.subcore_barrier()

        pltpu.sync_copy(
            acc.at[pl.ds(base, rows_per_tile)],
            out_hbm.at[cid].at[pl.ds(base, rows_per_tile)],
        )

    return seg(xt, col_p, row_p)


def kernel(x, edge_index, W, bias):
    n, d = x.shape
    e = edge_index.shape[1]

    blk = 1000
    grid = (n // blk,)
    xt = pl.pallas_call(
        _phase_a_body,
        grid=grid,
        in_specs=[
            pl.BlockSpec((blk, d), lambda i: (i, 0)),
            pl.BlockSpec((d, d), lambda i: (0, 0)),
            pl.BlockSpec((1, d), lambda i: (0, 0)),
        ],
        out_specs=pl.BlockSpec((blk, d), lambda i: (i, 0)),
        out_shape=jax.ShapeDtypeStruct((n, d), jnp.float32),
    )(x, W.T, bias.reshape(1, d))

    total_chunks = -(-e // _CH)
    pair_total = -(-total_chunks // _NUM_SUBCORES)
    nch1 = max(2, min(pair_total - 2, int(round(pair_total * 0.46))))
    nch0 = pair_total - nch1
    e_pad = _NUM_SUBCORES * pair_total * _CH
    n_pad = -(-(n + 1) // (8 * _NUM_SUBCORES)) * (8 * _NUM_SUBCORES)
    row = edge_index[0]
    col = edge_index[1]
    pad_dst = n + (jnp.arange(e_pad - e, dtype=jnp.int32) % (n_pad - n))
    col_flat = jnp.concatenate(
        [col, jnp.zeros((e_pad - e,), jnp.int32)]).reshape(-1, _CH)
    row_flat = jnp.concatenate([row, pad_dst]).reshape(-1, _CH)
    blocks = [nch0 if (w % _NUM_CORES) == 0 else nch1 for w in range(_NW)]
    starts = np.concatenate([[0], np.cumsum(blocks)[:-1]])
    nch_max = max(nch0, nch1)
    cmap = np.zeros((_NW, nch_max), np.int32)
    for w in range(_NW):
        cmap[w, :blocks[w]] = starts[w] + np.arange(blocks[w])
    cmap_j = jnp.asarray(cmap)
    col_p = col_flat[cmap_j]
    row_p = row_flat[cmap_j][:, :, None, :]

    parts = _sc_segment_sum(xt, col_p, row_p, n, n_pad, nch0, nch1)

    parts = parts[:, :n, :]
    out = pl.pallas_call(
        _phase_c_body,
        grid=grid,
        in_specs=[
            pl.BlockSpec((blk, d), lambda i: (i, 0)),
            pl.BlockSpec((blk, d), lambda i: (i, 0)),
        ],
        out_specs=pl.BlockSpec((blk, d), lambda i: (i, 0)),
        out_shape=jax.ShapeDtypeStruct((n, d), jnp.float32),
    )(parts[0], parts[1])
    return out


# speedup vs baseline: 1.3249x; 1.0130x over previous
import functools

import numpy as np

import jax
import jax.numpy as jnp
from jax import lax
from jax.experimental import pallas as pl
from jax.experimental.pallas import tpu as pltpu
from jax.experimental.pallas import tpu_sc as plsc

_EPS = 1e-15
_MAXNORM = 1.0 - 4e-3

_NUM_CORES = 2
_NUM_SUBCORES = 16
_NW = _NUM_CORES * _NUM_SUBCORES
_CH = 128
_SUP = 16


def _artanh(z):
    z = jnp.clip(z, -1.0 + 1e-7, 1.0 - 1e-7)
    return 0.5 * jnp.log((1.0 + z) / (1.0 - z))


def _row_norm(v):
    return jnp.maximum(jnp.sqrt(jnp.sum(v * v, axis=-1, keepdims=True)), _EPS)


def _proj(v):
    n = _row_norm(v)
    return jnp.where(n > _MAXNORM, v / n * _MAXNORM, v)


def _phase_a_body(x_ref, wt_ref, b_ref, o_ref):
    x = x_ref[...]
    mx = jnp.dot(x, wt_ref[...], preferred_element_type=jnp.float32)
    x_norm = _row_norm(x)
    mx_norm = _row_norm(mx)
    res = jnp.tanh(mx_norm / x_norm * _artanh(x_norm)) * mx / mx_norm
    res = jnp.where(jnp.all(mx == 0.0, axis=-1, keepdims=True), 0.0, res)
    h = _proj(res)
    b = b_ref[...]
    bn = _row_norm(b)
    y = _proj(jnp.tanh(bn) * b / bn)
    x2 = jnp.sum(h * h, axis=-1, keepdims=True)
    y2 = jnp.sum(y * y, axis=-1, keepdims=True)
    xy = jnp.sum(h * y, axis=-1, keepdims=True)
    num = (1.0 + 2.0 * xy + y2) * h + (1.0 - x2) * y
    den = 1.0 + 2.0 * xy + x2 * y2
    h = _proj(num / jnp.maximum(den, _EPS))
    pn = _row_norm(h)
    o_ref[...] = _artanh(pn) * h / pn


def _phase_c_body(a_ref, b_ref, o_ref):
    v = a_ref[...] + b_ref[...]
    sn = _row_norm(v)
    o1 = _proj(jnp.tanh(sn) * v / sn)
    pn = _row_norm(o1)
    xt = jnp.maximum(_artanh(pn) * o1 / pn, 0.0)
    un = _row_norm(xt)
    o_ref[...] = _proj(jnp.tanh(un) * xt / un)


def _sc_segment_sum(xt, col_p, row_p, n_nodes, n_pad, nch0, nch1):
    d = xt.shape[1]
    nch = max(nch0, nch1)
    mesh = plsc.VectorSubcoreMesh(core_axis_name="c", subcore_axis_name="s")

    @functools.partial(
        pl.kernel,
        mesh=mesh,
        out_type=jax.ShapeDtypeStruct((_NUM_CORES, n_pad, d), jnp.float32),
        scratch_types=[
            pltpu.VMEM((nch, _CH), jnp.int32),
            pltpu.VMEM((2, 1, _CH), jnp.int32),
            pltpu.VMEM((2, _CH, d), jnp.float32),
            pltpu.VMEM_SHARED((n_pad, d), jnp.float32),
            pltpu.SemaphoreType.DMA((2,)),
            pltpu.SemaphoreType.DMA((2,)),
        ],
    )
    def seg(xt_hbm, col_hbm, row_hbm, out_hbm, col_v, ridx, gbuf, acc,
            gsem, rsem):
        cid = lax.axis_index("c")
        sid = lax.axis_index("s")
        wid = sid * _NUM_CORES + cid
        nch_c = jnp.where(cid == 0, nch0, nch1)

        buf0 = gbuf.at[0]

        def _zero_row(i, carry):
            for k in range(d // 16):
                gbuf[0, i, pl.ds(k * 16, 16)] = jnp.zeros((16,), jnp.float32)
            return carry

        lax.fori_loop(0, _CH, _zero_row, 0)
        rows_per_tile = n_pad // _NUM_SUBCORES
        base = sid * rows_per_tile
        off = 0
        while off < rows_per_tile:
            nr = min(_CH, rows_per_tile - off)
            pltpu.sync_copy(buf0.at[pl.ds(0, nr)],
                            acc.at[pl.ds(base + off, nr)])
            off += nr

        pltpu.sync_copy(col_hbm.at[wid], col_v)
        pltpu.async_copy(row_hbm.at[wid].at[0], ridx.at[0], rsem.at[0])
        pltpu.async_copy(row_hbm.at[wid].at[1], ridx.at[1], rsem.at[1])
        plsc.subcore_barrier()

        pltpu.async_copy(xt_hbm.at[col_v.at[0]], gbuf.at[0], gsem.at[0])

        def _step(j, carry):
            b = lax.rem(j, 2)
            nb = 1 - b
            pltpu.make_async_copy(
                xt_hbm.at[col_v.at[j]], gbuf.at[b], gsem.at[b]).wait()

            @pl.when(j + 1 < nch_c)
            def _():
                pltpu.async_copy(
                    xt_hbm.at[col_v.at[j + 1]], gbuf.at[nb], gsem.at[nb])

            pltpu.make_async_copy(
                row_hbm.at[wid].at[j], ridx.at[b], rsem.at[b]).wait()
            pltpu.sync_copy(gbuf.at[b], acc.at[ridx.at[b, 0]], add=True)

            @pl.when(j + 2 < nch_c)
            def _():
                pltpu.async_copy(
                    row_hbm.at[wid].at[j + 2], ridx.at[b], rsem.at[b])

            return carry

        lax.fori_loop(0, nch_c, _step, 0)
        plsc.subcore_barrier()

        pltpu.sync_copy(
            acc.at[pl.ds(base, rows_per_tile)],
            out_hbm.at[cid].at[pl.ds(base, rows_per_tile)],
        )

    return seg(xt, col_p, row_p)


def kernel(x, edge_index, W, bias):
    n, d = x.shape
    e = edge_index.shape[1]

    blk = 1000
    grid = (n // blk,)
    xt = pl.pallas_call(
        _phase_a_body,
        grid=grid,
        in_specs=[
            pl.BlockSpec((blk, d), lambda i: (i, 0)),
            pl.BlockSpec((d, d), lambda i: (0, 0)),
            pl.BlockSpec((1, d), lambda i: (0, 0)),
        ],
        out_specs=pl.BlockSpec((blk, d), lambda i: (i, 0)),
        out_shape=jax.ShapeDtypeStruct((n, d), jnp.float32),
    )(x, W.T, bias.reshape(1, d))

    total_chunks = -(-e // _CH)
    pair_total = -(-total_chunks // _NUM_SUBCORES)
    nch1 = max(2, min(pair_total - 2, int(round(pair_total * 0.50))))
    nch0 = pair_total - nch1
    e_pad = _NUM_SUBCORES * pair_total * _CH
    n_pad = -(-(n + 1) // (8 * _NUM_SUBCORES)) * (8 * _NUM_SUBCORES)
    row = edge_index[0]
    col = edge_index[1]
    pad_dst = n + (jnp.arange(e_pad - e, dtype=jnp.int32) % (n_pad - n))
    col_flat = jnp.concatenate(
        [col, jnp.zeros((e_pad - e,), jnp.int32)]).reshape(-1, _CH)
    row_flat = jnp.concatenate([row, pad_dst]).reshape(-1, _CH)
    blocks = [nch0 if (w % _NUM_CORES) == 0 else nch1 for w in range(_NW)]
    starts = np.concatenate([[0], np.cumsum(blocks)[:-1]])
    nch_max = max(nch0, nch1)
    cmap = np.zeros((_NW, nch_max), np.int32)
    for w in range(_NW):
        cmap[w, :blocks[w]] = starts[w] + np.arange(blocks[w])
    cmap_j = jnp.asarray(cmap)
    col_p = col_flat[cmap_j]
    row_p = row_flat[cmap_j][:, :, None, :]

    parts = _sc_segment_sum(xt, col_p, row_p, n, n_pad, nch0, nch1)

    parts = parts[:, :n, :]
    out = pl.pallas_call(
        _phase_c_body,
        grid=grid,
        in_specs=[
            pl.BlockSpec((blk, d), lambda i: (i, 0)),
            pl.BlockSpec((blk, d), lambda i: (i, 0)),
        ],
        out_specs=pl.BlockSpec((blk, d), lambda i: (i, 0)),
        out_shape=jax.ShapeDtypeStruct((n, d), jnp.float32),
    )(parts[0], parts[1])
    return out
